# baseline (speedup 1.0000x reference)
import functools

import jax
import jax.numpy as jnp
from jax import lax
from jax.experimental import pallas as pl
from jax.experimental.pallas import tpu as pltpu
from jax.experimental.pallas import tpu_sc as plsc

NC = 2
NS = 16
L = 16
NW = NC * NS
B = 16384
K = 32
BPW = B // NW
NCHUNK = BPW // 128
NG = BPW // L


def _lfm_body(sampleU_h, sampleI_h, sampleR_h, alpha_h, betaU_h, betaI_h,
              gammaU_h, gammaI_h, out_h,
              idxU_v, idxI_v, rowsU_v, rowsI_v, bu_v, bi_v, r_v, alpha_v,
              acc_v, sem):
    c = lax.axis_index("c")
    s = lax.axis_index("s")
    wid = s * NC + c
    base = wid * BPW

    pltpu.sync_copy(sampleU_h.at[pl.ds(base, BPW)], idxU_v)
    pltpu.sync_copy(sampleI_h.at[pl.ds(base, BPW)], idxI_v)

    copies = []
    for j in range(NCHUNK):
        dst = pl.ds(j * 128, 128)
        copies.append(pltpu.async_copy(
            gammaU_h.at[idxU_v.at[dst]], rowsU_v.at[dst], sem))
        copies.append(pltpu.async_copy(
            gammaI_h.at[idxI_v.at[dst]], rowsI_v.at[dst], sem))
        copies.append(pltpu.async_copy(
            betaU_h.at[idxU_v.at[dst]], bu_v.at[dst], sem))
        copies.append(pltpu.async_copy(
            betaI_h.at[idxI_v.at[dst]], bi_v.at[dst], sem))
    copies.append(pltpu.async_copy(
        sampleR_h.at[pl.ds(base, BPW)], r_v, sem))
    copies.append(pltpu.async_copy(alpha_h, alpha_v, sem))
    for cp in copies:
        cp.wait()

    alpha_vec = alpha_v[...]
    lanes = lax.iota(jnp.int32, L)

    def group_body(g, acc):
        gbase = g * L
        row_ids = gbase + lanes

        def k_body(k, dots):
            col = jnp.zeros((L,), jnp.int32) + k
            gu = plsc.load_gather(rowsU_v, [row_ids, col])
            gi = plsc.load_gather(rowsI_v, [row_ids, col])
            return dots + gu * gi

        dots = lax.fori_loop(0, K, k_body, jnp.zeros((L,), jnp.float32))
        bu = bu_v[pl.ds(gbase, L)]
        bi = bi_v[pl.ds(gbase, L)]
        r = r_v[pl.ds(gbase, L)]
        diff = alpha_vec + bu + bi + dots - r
        return acc + diff * diff

    acc = lax.fori_loop(0, NG, group_body, jnp.zeros((L,), jnp.float32))
    acc_v[...] = acc
    pltpu.sync_copy(acc_v, out_h.at[pl.ds(wid * L, L)])


@jax.jit
def _lfm(sampleU, sampleI, sampleR, alpha16, betaU, betaI, gammaU, gammaI):
    mesh = plsc.VectorSubcoreMesh(core_axis_name="c", subcore_axis_name="s")
    kern = functools.partial(
        pl.kernel, mesh=mesh,
        out_type=jax.ShapeDtypeStruct((NW * L,), jnp.float32),
        scratch_types=[
            pltpu.VMEM((BPW,), jnp.int32),
            pltpu.VMEM((BPW,), jnp.int32),
            pltpu.VMEM((BPW, K), jnp.float32),
            pltpu.VMEM((BPW, K), jnp.float32),
            pltpu.VMEM((BPW,), jnp.float32),
            pltpu.VMEM((BPW,), jnp.float32),
            pltpu.VMEM((BPW,), jnp.float32),
            pltpu.VMEM((L,), jnp.float32),
            pltpu.VMEM((L,), jnp.float32),
            pltpu.SemaphoreType.DMA,
        ],
        compiler_params=pltpu.CompilerParams(
            needs_layout_passes=False, use_tc_tiling_on_sc=False),
    )(_lfm_body)
    return kern(sampleU, sampleI, sampleR, alpha16, betaU, betaI,
                gammaU, gammaI)


def kernel(sampleU, sampleI, sampleR, alpha, betaU, betaI, gammaU, gammaI):
    alpha16 = jnp.broadcast_to(alpha, (L,)).astype(jnp.float32)
    partials = _lfm(sampleU, sampleI, sampleR, alpha16,
                    betaU, betaI, gammaU, gammaI)
    return 0.5 * jnp.sum(partials) / sampleR.shape[0]


# speedup vs baseline: 2.0912x; 2.0912x over previous
import functools

import jax
import jax.numpy as jnp
from jax import lax
from jax.experimental import pallas as pl
from jax.experimental.pallas import tpu as pltpu
from jax.experimental.pallas import tpu_sc as plsc

NC = 2
NS = 16
L = 16
NW = NC * NS
B = 16384
K = 32
BPW = B // NW
NG = BPW // L


def _lfm_body(sampleU_h, sampleI_h, sampleR_h, alpha_h, betaU_h, betaI_h,
              gammaU_h, gammaI_h, out_h,
              idxU_v, idxI_v, rowsU_v, rowsI_v,
              bu_v, bi_v, r_v, alpha_v, acc_v, sem, gsem):
    c = lax.axis_index("c")
    s = lax.axis_index("s")
    wid = s * NC + c
    base = wid * BPW

    pltpu.sync_copy(sampleU_h.at[pl.ds(base, BPW)], idxU_v)
    pltpu.sync_copy(sampleI_h.at[pl.ds(base, BPW)], idxI_v)

    copies = []
    for j in range(BPW // 128):
        dst = pl.ds(j * 128, 128)
        copies.append(pltpu.async_copy(
            betaU_h.at[idxU_v.at[dst]], bu_v.at[dst], sem))
        copies.append(pltpu.async_copy(
            betaI_h.at[idxI_v.at[dst]], bi_v.at[dst], sem))
    copies.append(pltpu.async_copy(
        sampleR_h.at[pl.ds(base, BPW)], r_v, sem))
    copies.append(pltpu.async_copy(alpha_h, alpha_v, sem))

    lanes = lax.iota(jnp.int32, L)

    def fire(g):
        par = g % 2
        gbase = g * L
        vu = lax.shift_right_logical(idxU_v[pl.ds(gbase, L)], 3)
        vi = lax.shift_right_logical(idxI_v[pl.ds(gbase, L)], 3)
        for j in range(L):
            pltpu.async_copy(gammaU_h.at[pl.ds(vu[j], 1)],
                             rowsU_v.at[par].at[pl.ds(j, 1)], gsem)
            pltpu.async_copy(gammaI_h.at[pl.ds(vi[j], 1)],
                             rowsI_v.at[par].at[pl.ds(j, 1)], gsem)
        return None, None

    def compute(g):
        par = g % 2
        gbase = g * L
        vu = idxU_v[pl.ds(gbase, L)]
        vi = idxI_v[pl.ds(gbase, L)]
        su = lax.bitwise_and(vu, jnp.int32(7))
        si = lax.bitwise_and(vi, jnp.int32(7))
        pvec = jnp.zeros((L,), jnp.int32) + par

        def k_body(k, dots):
            kvec = jnp.zeros((L,), jnp.int32) + k
            gu = plsc.load_gather(rowsU_v, [pvec, lanes, su, kvec])
            gi = plsc.load_gather(rowsI_v, [pvec, lanes, si, kvec])
            return dots + gu * gi

        dots = lax.fori_loop(0, K, k_body, jnp.zeros((L,), jnp.float32))
        bu = bu_v[pl.ds(gbase, L)]
        bi = bi_v[pl.ds(gbase, L)]
        r = r_v[pl.ds(gbase, L)]
        diff = alpha_v[...] + bu + bi + dots - r
        return diff * diff

    for cp in copies:
        cp.wait()

    def drain_round(par):
        pltpu.make_async_copy(
            gammaU_h.at[pl.ds(0, L)], rowsU_v.at[par], gsem).wait()
        pltpu.make_async_copy(
            gammaI_h.at[pl.ds(0, L)], rowsI_v.at[par], gsem).wait()

    fire(0)
    drain_round(0)

    def loop_body(g, acc):
        fire(g + 1)
        res = compute(g)
        drain_round((g + 1) % 2)
        return acc + res

    acc = lax.fori_loop(0, NG - 1, loop_body, jnp.zeros((L,), jnp.float32))
    acc = acc + compute(NG - 1)
    acc_v[...] = acc
    pltpu.sync_copy(acc_v, out_h.at[pl.ds(wid * L, L)])


@jax.jit
def _lfm(sampleU, sampleI, sampleR, alpha16, betaU, betaI, gammaU3, gammaI3):
    mesh = plsc.VectorSubcoreMesh(core_axis_name="c", subcore_axis_name="s")
    kern = functools.partial(
        pl.kernel, mesh=mesh,
        out_type=jax.ShapeDtypeStruct((NW * L,), jnp.float32),
        scratch_types=[
            pltpu.VMEM((BPW,), jnp.int32),
            pltpu.VMEM((BPW,), jnp.int32),
            pltpu.VMEM((2, L, 8, K), jnp.float32),
            pltpu.VMEM((2, L, 8, K), jnp.float32),
            pltpu.VMEM((BPW,), jnp.float32),
            pltpu.VMEM((BPW,), jnp.float32),
            pltpu.VMEM((BPW,), jnp.float32),
            pltpu.VMEM((L,), jnp.float32),
            pltpu.VMEM((L,), jnp.float32),
            pltpu.SemaphoreType.DMA,
            pltpu.SemaphoreType.DMA,
        ],
        compiler_params=pltpu.CompilerParams(needs_layout_passes=False),
    )(_lfm_body)
    return kern(sampleU, sampleI, sampleR, alpha16, betaU, betaI,
                gammaU3, gammaI3)


def kernel(sampleU, sampleI, sampleR, alpha, betaU, betaI, gammaU, gammaI):
    alpha16 = jnp.broadcast_to(alpha, (L,)).astype(jnp.float32)
    gammaU3 = gammaU.reshape(gammaU.shape[0] // 8, 8, K)
    gammaI3 = gammaI.reshape(gammaI.shape[0] // 8, 8, K)
    partials = _lfm(sampleU, sampleI, sampleR, alpha16,
                    betaU, betaI, gammaU3, gammaI3)
    return 0.5 * jnp.sum(partials) / sampleR.shape[0]


# speedup vs baseline: 2.1010x; 1.0047x over previous
import functools

import jax
import jax.numpy as jnp
from jax import lax
from jax.experimental import pallas as pl
from jax.experimental.pallas import tpu as pltpu
from jax.experimental.pallas import tpu_sc as plsc

NC = 2
NS = 16
L = 16
NW = NC * NS
B = 16384
K = 32
BPW = B // NW
NG = BPW // L


def _lfm_body(sampleU_h, sampleI_h, sampleR_h, alpha_h, betaU_h, betaI_h,
              gammaU_h, gammaI_h, out_h,
              idxU_v, idxI_v, rowsU_v, rowsI_v,
              bu_v, bi_v, r_v, alpha_v, acc_v, sem, gsem):
    c = lax.axis_index("c")
    s = lax.axis_index("s")
    wid = s * NC + c
    base = wid * BPW

    pltpu.sync_copy(sampleU_h.at[pl.ds(base, BPW)], idxU_v)
    pltpu.sync_copy(sampleI_h.at[pl.ds(base, BPW)], idxI_v)

    copies = []
    for j in range(BPW // 128):
        dst = pl.ds(j * 128, 128)
        copies.append(pltpu.async_copy(
            betaU_h.at[idxU_v.at[dst]], bu_v.at[dst], sem))
        copies.append(pltpu.async_copy(
            betaI_h.at[idxI_v.at[dst]], bi_v.at[dst], sem))
    copies.append(pltpu.async_copy(
        sampleR_h.at[pl.ds(base, BPW)], r_v, sem))
    copies.append(pltpu.async_copy(alpha_h, alpha_v, sem))

    lanes = lax.iota(jnp.int32, L)

    def fire(g):
        par = g % 2
        gbase = g * L
        vu = lax.shift_right_logical(idxU_v[pl.ds(gbase, L)], 3)
        vi = lax.shift_right_logical(idxI_v[pl.ds(gbase, L)], 3)
        for j in range(L):
            pltpu.async_copy(gammaU_h.at[pl.ds(vu[j], 1)],
                             rowsU_v.at[par].at[pl.ds(j, 1)], gsem)
            pltpu.async_copy(gammaI_h.at[pl.ds(vi[j], 1)],
                             rowsI_v.at[par].at[pl.ds(j, 1)], gsem)
        return None, None

    def compute(g):
        par = g % 2
        gbase = g * L
        vu = idxU_v[pl.ds(gbase, L)]
        vi = idxI_v[pl.ds(gbase, L)]
        su = lax.bitwise_and(vu, jnp.int32(7))
        si = lax.bitwise_and(vi, jnp.int32(7))
        pvec = jnp.zeros((L,), jnp.int32) + par

        def k_body(k, dots):
            kvec = jnp.zeros((L,), jnp.int32) + k
            gu = plsc.load_gather(rowsU_v, [pvec, lanes, su, kvec])
            gi = plsc.load_gather(rowsI_v, [pvec, lanes, si, kvec])
            return dots + gu * gi

        dots = lax.fori_loop(0, K, k_body, jnp.zeros((L,), jnp.float32))
        bu = bu_v[pl.ds(gbase, L)]
        bi = bi_v[pl.ds(gbase, L)]
        r = r_v[pl.ds(gbase, L)]
        diff = alpha_v[...] + bu + bi + dots - r
        return diff * diff

    for cp in copies:
        cp.wait()

    def drain_round(par):
        pltpu.make_async_copy(
            gammaU_h.at[pl.ds(0, L)], rowsU_v.at[par], gsem).wait()
        pltpu.make_async_copy(
            gammaI_h.at[pl.ds(0, L)], rowsI_v.at[par], gsem).wait()

    fire(0)
    drain_round(0)

    def loop_body(g, acc):
        fire(g + 1)
        res = compute(g)
        drain_round((g + 1) % 2)
        return acc + res

    acc = lax.fori_loop(0, NG - 1, loop_body, jnp.zeros((L,), jnp.float32))
    acc = acc + compute(NG - 1)
    acc_v[...] = acc
    pltpu.sync_copy(acc_v, out_h.at[pl.ds(wid * L, L)])


@jax.jit
def _lfm(sampleU, sampleI, sampleR, alpha16, betaU, betaI, gammaU3, gammaI3):
    mesh = plsc.VectorSubcoreMesh(core_axis_name="c", subcore_axis_name="s")
    kern = functools.partial(
        pl.kernel, mesh=mesh,
        out_type=jax.ShapeDtypeStruct((NW * L,), jnp.float32),
        scratch_types=[
            pltpu.VMEM((BPW,), jnp.int32),
            pltpu.VMEM((BPW,), jnp.int32),
            pltpu.VMEM((2, L, 8, K), jnp.float32),
            pltpu.VMEM((2, L, 8, K), jnp.float32),
            pltpu.VMEM((BPW,), jnp.float32),
            pltpu.VMEM((BPW,), jnp.float32),
            pltpu.VMEM((BPW,), jnp.float32),
            pltpu.VMEM((L,), jnp.float32),
            pltpu.VMEM((L,), jnp.float32),
            pltpu.SemaphoreType.DMA,
            pltpu.SemaphoreType.DMA,
        ],
        compiler_params=pltpu.CompilerParams(
            needs_layout_passes=False, use_tc_tiling_on_sc=True),
    )(_lfm_body)
    return kern(sampleU, sampleI, sampleR, alpha16, betaU, betaI,
                gammaU3, gammaI3)


def kernel(sampleU, sampleI, sampleR, alpha, betaU, betaI, gammaU, gammaI):
    alpha16 = jnp.broadcast_to(alpha, (L,)).astype(jnp.float32)
    gammaU3 = gammaU.reshape(gammaU.shape[0] // 8, 8, K)
    gammaI3 = gammaI.reshape(gammaI.shape[0] // 8, 8, K)
    partials = _lfm(sampleU, sampleI, sampleR, alpha16,
                    betaU, betaI, gammaU3, gammaI3)
    return 0.5 * jnp.sum(partials) / sampleR.shape[0]
